# baseline (speedup 1.0000x reference)
import functools

import jax
import jax.numpy as jnp
from jax import lax
from jax.experimental import pallas as pl
from jax.experimental.pallas import tpu as pltpu
from jax.experimental.pallas import tpu_sc as plsc

N_NODES = 10000
N_EDGES = 320000
F_IN = 128
HID = 128
C_OUT = 64

NC = 2
NS = 16
NW = NC * NS

CHUNK = 80
EDGES_PER_TILE = N_EDGES // NW
CH_PER_TILE = EDGES_PER_TILE // CHUNK
N_ACC = N_NODES
ROWS_OUT = N_ACC // NS
W1 = 128
W2 = 64

NBUF = 2
BLK_M = 1024
GRID_M = 10


def _make_sc_pass(width, with_deg):
    mesh = plsc.VectorSubcoreMesh(core_axis_name="c", subcore_axis_name="s")

    out_type = [jax.ShapeDtypeStruct((NC, N_ACC, width), jnp.float32)]
    scratch = [
        pltpu.VMEM((CH_PER_TILE, CHUNK), jnp.int32),
        pltpu.VMEM((CH_PER_TILE, CHUNK), jnp.int32),
        pltpu.VMEM((CHUNK, width), jnp.float32),
        pltpu.VMEM((CHUNK, width), jnp.float32),
        pltpu.VMEM_SHARED((N_ACC, width), jnp.float32),
    ]
    if with_deg:
        out_type.append(jax.ShapeDtypeStruct((NC, N_ACC, 8), jnp.float32))
        scratch += [
            pltpu.VMEM((CHUNK, 8), jnp.float32),
            pltpu.VMEM_SHARED((N_ACC, 8), jnp.float32),
        ]
    scratch += [pltpu.SemaphoreType.DMA] * (2 * NBUF)

    @functools.partial(
        pl.kernel,
        out_type=out_type,
        mesh=mesh,
        compiler_params=pltpu.CompilerParams(use_tc_tiling_on_sc=False),
        scratch_types=scratch,
    )
    def sc_pass(table, adj4, zeros, *rest):
        if with_deg:
            zd, ones, out, outd, src_v, dst_v, r0, r1, acc, ones_v, accd = \
                rest[:11]
            sems = rest[11:]
        else:
            out, src_v, dst_v, r0, r1, acc = rest[:6]
            sems = rest[6:]
        rows = (r0, r1)
        gs = sems[:NBUF]
        ss = sems[NBUF:]
        c = lax.axis_index("c")
        s = lax.axis_index("s")
        wid = c * NS + s
        pltpu.sync_copy(zeros, acc.at[pl.ds(s * ROWS_OUT, ROWS_OUT)])
        if with_deg:
            pltpu.sync_copy(zd, accd.at[pl.ds(s * ROWS_OUT, ROWS_OUT)])
            pltpu.sync_copy(ones, ones_v)
        pltpu.sync_copy(adj4.at[0, wid], src_v)
        pltpu.sync_copy(adj4.at[1, wid], dst_v)
        plsc.subcore_barrier()

        def gi(j, b):
            pltpu.async_copy(table.at[src_v.at[j]], rows[b], gs[b])

        def gw(b):
            pltpu.make_async_copy(table.at[src_v.at[0]], rows[b],
                                  gs[b]).wait()

        def si(j, b):
            pltpu.async_copy(rows[b], acc.at[dst_v.at[j]], ss[b], add=True)
            if with_deg:
                pltpu.async_copy(ones_v, accd.at[dst_v.at[j]], ss[b],
                                 add=True)

        def sw(b):
            pltpu.make_async_copy(rows[b], acc.at[dst_v.at[0]],
                                  ss[b]).wait()
            if with_deg:
                pltpu.make_async_copy(ones_v, accd.at[dst_v.at[0]],
                                      ss[b]).wait()

        gi(0, 0)
        gi(1, 1)

        @pl.loop(0, (CH_PER_TILE - 3) // 2)
        def _(i):
            j0 = i * 2
            for b in range(2):
                j = j0 + b
                gw(b)
                si(j, b)
                sw(b)
                gi(j + 2, b)

        e = CH_PER_TILE - 3
        gw(0); si(e, 0); sw(0); gi(e + 2, 0)
        gw(1); si(e + 1, 1); sw(1)
        gw(0); si(e + 2, 0); sw(0)

        plsc.subcore_barrier()
        pltpu.sync_copy(
            acc.at[pl.ds(s * ROWS_OUT, ROWS_OUT)],
            out.at[c, pl.ds(s * ROWS_OUT, ROWS_OUT)],
        )
        if with_deg:
            pltpu.sync_copy(
                accd.at[pl.ds(s * ROWS_OUT, ROWS_OUT)],
                outd.at[c, pl.ds(s * ROWS_OUT, ROWS_OUT)],
            )

    return sc_pass


_sc_pass1 = _make_sc_pass(W1, True)
_sc_pass2 = _make_sc_pass(W2, False)


def _tc_mid_body(x_ref, p0_ref, p1_ref, degp0_ref, degp1_ref, w1s_ref,
                 w1n_ref, b1_ref, w2n_ref, w2s_ref, b2_ref, hn_ref, hs_ref,
                 dinv_ref):
    p = p0_ref[0] + p1_ref[0]
    deg = degp0_ref[0, :, 0:1] + degp1_ref[0, :, 0:1]
    dinv = 1.0 / jnp.maximum(deg, 1.0)
    mean = p * dinv
    h = x_ref[...] @ w1s_ref[...] + mean @ w1n_ref[...] + b1_ref[...]
    h = jnp.maximum(h, 0.0)
    hn_ref[...] = h @ w2n_ref[...]
    hs_ref[...] = h @ w2s_ref[...] + b2_ref[...]
    dinv_ref[...] = dinv


def _tc_fin_body(hs_ref, q0_ref, q1_ref, dinv_ref, out_ref):
    agg = q0_ref[0] + q1_ref[0]
    out_ref[...] = hs_ref[...] + agg * dinv_ref[...]


_tc_mid = pl.pallas_call(
    _tc_mid_body,
    grid=(GRID_M,),
    in_specs=[
        pl.BlockSpec((BLK_M, F_IN), lambda i: (i, 0)),
        pl.BlockSpec((1, BLK_M, W1), lambda i: (0, i, 0)),
        pl.BlockSpec((1, BLK_M, W1), lambda i: (1, i, 0)),
        pl.BlockSpec((1, BLK_M, 8), lambda i: (0, i, 0)),
        pl.BlockSpec((1, BLK_M, 8), lambda i: (1, i, 0)),
        pl.BlockSpec((F_IN, HID), lambda i: (0, 0)),
        pl.BlockSpec((F_IN, HID), lambda i: (0, 0)),
        pl.BlockSpec((1, HID), lambda i: (0, 0)),
        pl.BlockSpec((HID, C_OUT), lambda i: (0, 0)),
        pl.BlockSpec((HID, C_OUT), lambda i: (0, 0)),
        pl.BlockSpec((1, C_OUT), lambda i: (0, 0)),
    ],
    out_specs=[
        pl.BlockSpec((BLK_M, W2), lambda i: (i, 0)),
        pl.BlockSpec((BLK_M, C_OUT), lambda i: (i, 0)),
        pl.BlockSpec((BLK_M, 1), lambda i: (i, 0)),
    ],
    out_shape=[
        jax.ShapeDtypeStruct((N_NODES, W2), jnp.float32),
        jax.ShapeDtypeStruct((N_NODES, C_OUT), jnp.float32),
        jax.ShapeDtypeStruct((N_NODES, 1), jnp.float32),
    ],
)

_tc_fin = pl.pallas_call(
    _tc_fin_body,
    grid=(GRID_M,),
    in_specs=[
        pl.BlockSpec((BLK_M, C_OUT), lambda i: (i, 0)),
        pl.BlockSpec((1, BLK_M, C_OUT), lambda i: (0, i, 0)),
        pl.BlockSpec((1, BLK_M, C_OUT), lambda i: (1, i, 0)),
        pl.BlockSpec((BLK_M, 1), lambda i: (i, 0)),
    ],
    out_specs=pl.BlockSpec((BLK_M, C_OUT), lambda i: (i, 0)),
    out_shape=jax.ShapeDtypeStruct((N_NODES, C_OUT), jnp.float32),
)


@jax.jit
def kernel(x, adj, W1_self, W1_neigh, b1, W2_self, W2_neigh, b2):
    adj4 = adj.reshape(2, NW, CH_PER_TILE, CHUNK)

    z1 = jnp.zeros((ROWS_OUT, W1), jnp.float32)
    z2 = jnp.zeros((ROWS_OUT, W2), jnp.float32)
    zd = jnp.zeros((ROWS_OUT, 8), jnp.float32)
    ones = jnp.ones((CHUNK, 8), jnp.float32)

    parts1, degp = jax.tree_util.tree_leaves(
        _sc_pass1(x, adj4, z1, zd, ones))
    hn, hs, dinv = _tc_mid(x, parts1, parts1, degp, degp, W1_self,
                           W1_neigh, b1.reshape(1, HID), W2_neigh, W2_self,
                           b2.reshape(1, C_OUT))
    parts2 = jax.tree_util.tree_leaves(_sc_pass2(hn, adj4, z2))[0]
    return _tc_fin(hs, parts2, parts2, dinv)


# speedup vs baseline: 1.0103x; 1.0103x over previous
import functools

import jax
import jax.numpy as jnp
from jax import lax
from jax.experimental import pallas as pl
from jax.experimental.pallas import tpu as pltpu
from jax.experimental.pallas import tpu_sc as plsc

N_NODES = 10000
N_EDGES = 320000
F_IN = 128
HID = 128
C_OUT = 64

NC = 2
NS = 16
NW = NC * NS

CHUNK = 80
EDGES_PER_TILE = N_EDGES // NW
CH_PER_TILE = EDGES_PER_TILE // CHUNK
N_ACC = N_NODES
ROWS_OUT = N_ACC // NS
W1 = 128
W2 = 64

NBUF = 2
BLK_M = 2048
GRID_M = 5


def _make_sc_pass(width, with_deg):
    mesh = plsc.VectorSubcoreMesh(core_axis_name="c", subcore_axis_name="s")

    out_type = [jax.ShapeDtypeStruct((NC, N_ACC, width), jnp.float32)]
    scratch = [
        pltpu.VMEM((CH_PER_TILE, CHUNK), jnp.int32),
        pltpu.VMEM((CH_PER_TILE, CHUNK), jnp.int32),
        pltpu.VMEM((CHUNK, width), jnp.float32),
        pltpu.VMEM((CHUNK, width), jnp.float32),
        pltpu.VMEM_SHARED((N_ACC, width), jnp.float32),
    ]
    if with_deg:
        out_type.append(jax.ShapeDtypeStruct((NC, N_ACC, 8), jnp.float32))
        scratch += [
            pltpu.VMEM((CHUNK, 8), jnp.float32),
            pltpu.VMEM_SHARED((N_ACC, 8), jnp.float32),
        ]
    scratch += [pltpu.SemaphoreType.DMA] * (2 * NBUF)

    @functools.partial(
        pl.kernel,
        out_type=out_type,
        mesh=mesh,
        compiler_params=pltpu.CompilerParams(use_tc_tiling_on_sc=False),
        scratch_types=scratch,
    )
    def sc_pass(table, adj4, zeros, *rest):
        if with_deg:
            zd, ones, out, outd, src_v, dst_v, r0, r1, acc, ones_v, accd = \
                rest[:11]
            sems = rest[11:]
        else:
            out, src_v, dst_v, r0, r1, acc = rest[:6]
            sems = rest[6:]
        rows = (r0, r1)
        gs = sems[:NBUF]
        ss = sems[NBUF:]
        c = lax.axis_index("c")
        s = lax.axis_index("s")
        wid = c * NS + s
        pltpu.sync_copy(zeros, acc.at[pl.ds(s * ROWS_OUT, ROWS_OUT)])
        if with_deg:
            pltpu.sync_copy(zd, accd.at[pl.ds(s * ROWS_OUT, ROWS_OUT)])
            pltpu.sync_copy(ones, ones_v)
        pltpu.sync_copy(adj4.at[0, wid], src_v)
        pltpu.sync_copy(adj4.at[1, wid], dst_v)
        plsc.subcore_barrier()

        def gi(j, b):
            pltpu.async_copy(table.at[src_v.at[j]], rows[b], gs[b])

        def gw(b):
            pltpu.make_async_copy(table.at[src_v.at[0]], rows[b],
                                  gs[b]).wait()

        def si(j, b):
            pltpu.async_copy(rows[b], acc.at[dst_v.at[j]], ss[b], add=True)
            if with_deg:
                pltpu.async_copy(ones_v, accd.at[dst_v.at[j]], ss[b],
                                 add=True)

        def sw(b):
            pltpu.make_async_copy(rows[b], acc.at[dst_v.at[0]],
                                  ss[b]).wait()
            if with_deg:
                pltpu.make_async_copy(ones_v, accd.at[dst_v.at[0]],
                                      ss[b]).wait()

        gi(0, 0)
        gi(1, 1)

        @pl.loop(0, (CH_PER_TILE - 3) // 2)
        def _(i):
            j0 = i * 2
            for b in range(2):
                j = j0 + b
                gw(b)
                si(j, b)
                sw(b)
                gi(j + 2, b)

        e = CH_PER_TILE - 3
        gw(0); si(e, 0); sw(0); gi(e + 2, 0)
        gw(1); si(e + 1, 1); sw(1)
        gw(0); si(e + 2, 0); sw(0)

        plsc.subcore_barrier()
        pltpu.sync_copy(
            acc.at[pl.ds(s * ROWS_OUT, ROWS_OUT)],
            out.at[c, pl.ds(s * ROWS_OUT, ROWS_OUT)],
        )
        if with_deg:
            pltpu.sync_copy(
                accd.at[pl.ds(s * ROWS_OUT, ROWS_OUT)],
                outd.at[c, pl.ds(s * ROWS_OUT, ROWS_OUT)],
            )

    return sc_pass


_sc_pass1 = _make_sc_pass(W1, True)
_sc_pass2 = _make_sc_pass(W2, False)


def _tc_mid_body(x_ref, p0_ref, p1_ref, degp0_ref, degp1_ref, w1s_ref,
                 w1n_ref, b1_ref, w2n_ref, w2s_ref, b2_ref, hn_ref, hs_ref,
                 dinv_ref):
    p = p0_ref[0] + p1_ref[0]
    deg = degp0_ref[0, :, 0:1] + degp1_ref[0, :, 0:1]
    dinv = 1.0 / jnp.maximum(deg, 1.0)
    mean = p * dinv
    h = x_ref[...] @ w1s_ref[...] + mean @ w1n_ref[...] + b1_ref[...]
    h = jnp.maximum(h, 0.0)
    hn_ref[...] = h @ w2n_ref[...]
    hs_ref[...] = h @ w2s_ref[...] + b2_ref[...]
    dinv_ref[...] = dinv


def _tc_fin_body(hs_ref, q0_ref, q1_ref, dinv_ref, out_ref):
    agg = q0_ref[0] + q1_ref[0]
    out_ref[...] = hs_ref[...] + agg * dinv_ref[...]


_tc_mid = pl.pallas_call(
    _tc_mid_body,
    grid=(GRID_M,),
    in_specs=[
        pl.BlockSpec((BLK_M, F_IN), lambda i: (i, 0)),
        pl.BlockSpec((1, BLK_M, W1), lambda i: (0, i, 0)),
        pl.BlockSpec((1, BLK_M, W1), lambda i: (1, i, 0)),
        pl.BlockSpec((1, BLK_M, 8), lambda i: (0, i, 0)),
        pl.BlockSpec((1, BLK_M, 8), lambda i: (1, i, 0)),
        pl.BlockSpec((F_IN, HID), lambda i: (0, 0)),
        pl.BlockSpec((F_IN, HID), lambda i: (0, 0)),
        pl.BlockSpec((1, HID), lambda i: (0, 0)),
        pl.BlockSpec((HID, C_OUT), lambda i: (0, 0)),
        pl.BlockSpec((HID, C_OUT), lambda i: (0, 0)),
        pl.BlockSpec((1, C_OUT), lambda i: (0, 0)),
    ],
    out_specs=[
        pl.BlockSpec((BLK_M, W2), lambda i: (i, 0)),
        pl.BlockSpec((BLK_M, C_OUT), lambda i: (i, 0)),
        pl.BlockSpec((BLK_M, 1), lambda i: (i, 0)),
    ],
    out_shape=[
        jax.ShapeDtypeStruct((N_NODES, W2), jnp.float32),
        jax.ShapeDtypeStruct((N_NODES, C_OUT), jnp.float32),
        jax.ShapeDtypeStruct((N_NODES, 1), jnp.float32),
    ],
)

_tc_fin = pl.pallas_call(
    _tc_fin_body,
    grid=(GRID_M,),
    in_specs=[
        pl.BlockSpec((BLK_M, C_OUT), lambda i: (i, 0)),
        pl.BlockSpec((1, BLK_M, C_OUT), lambda i: (0, i, 0)),
        pl.BlockSpec((1, BLK_M, C_OUT), lambda i: (1, i, 0)),
        pl.BlockSpec((BLK_M, 1), lambda i: (i, 0)),
    ],
    out_specs=pl.BlockSpec((BLK_M, C_OUT), lambda i: (i, 0)),
    out_shape=jax.ShapeDtypeStruct((N_NODES, C_OUT), jnp.float32),
)


@jax.jit
def kernel(x, adj, W1_self, W1_neigh, b1, W2_self, W2_neigh, b2):
    adj4 = adj.reshape(2, NW, CH_PER_TILE, CHUNK)

    z1 = jnp.zeros((ROWS_OUT, W1), jnp.float32)
    z2 = jnp.zeros((ROWS_OUT, W2), jnp.float32)
    zd = jnp.zeros((ROWS_OUT, 8), jnp.float32)
    ones = jnp.ones((CHUNK, 8), jnp.float32)

    parts1, degp = jax.tree_util.tree_leaves(
        _sc_pass1(x, adj4, z1, zd, ones))
    hn, hs, dinv = _tc_mid(x, parts1, parts1, degp, degp, W1_self,
                           W1_neigh, b1.reshape(1, HID), W2_neigh, W2_self,
                           b2.reshape(1, C_OUT))
    parts2 = jax.tree_util.tree_leaves(_sc_pass2(hn, adj4, z2))[0]
    return _tc_fin(hs, parts2, parts2, dinv)
